# baseline (speedup 1.0000x reference)
import jax
import jax.numpy as jnp
from jax import lax
from jax.experimental import pallas as pl
from jax.experimental.pallas import tpu as pltpu
from jax.experimental.pallas import tpu_sc as plsc

NC, NS, L = 2, 16, 16
NW = NC * NS
B, S, H = 32, 1024, 1024
SW = S // NW
R = 8
CPB = SW // R


def _emb_body(x_hbm, pix_hbm, pos_hbm, out_hbm,
              idx_v, pos_v, g0, g1, g2, g3, o0, o1, o2, o3,
              gsem0, gsem1, gsem2, gsem3, stsem0, stsem1, stsem2, stsem3):
    wid = lax.axis_index("s") * NC + lax.axis_index("c")
    col0 = pl.multiple_of(wid * SW, SW)
    xblk = pl.multiple_of((wid // 4) * 128, 128)
    coff = (wid % 4) * SW
    G = (g0, g1, g2, g3)
    O = (o0, o1, o2, o3)
    GSEM = (gsem0, gsem1, gsem2, gsem3)
    STSEM = (stsem0, stsem1, stsem2, stsem3)

    pltpu.sync_copy(x_hbm.at[:, pl.ds(xblk, 128)], idx_v)
    pos_copy = pltpu.async_copy(pos_hbm.at[pl.ds(col0, SW), :], pos_v, stsem0)

    def start_gather(b, j):
        pltpu.async_copy(
            pix_hbm.at[idx_v.at[b, pl.ds(coff + j * R, R)]], G[j], GSEM[j]
        )

    def wait_gather(b, j):
        pltpu.make_async_copy(
            pix_hbm.at[idx_v.at[b, pl.ds(coff + j * R, R)]], G[j], GSEM[j]
        ).wait()

    def wait_store(b, j):
        pltpu.make_async_copy(
            O[j], out_hbm.at[b, pl.ds(col0 + j * R, R), :], STSEM[j]
        ).wait()

    for j in range(CPB):
        start_gather(0, j)
    pos_copy.wait()

    def step(b, carry):
        for j in range(CPB):
            srow = j * R
            gbuf, obuf = G[j], O[j]
            wait_gather(b, j)
            @pl.when(b > 0)
            def _():
                wait_store(b, j)
            @plsc.parallel_loop(0, H // L, step=1, unroll=2)
            def _(u, _obuf=obuf, _gbuf=gbuf, _srow=srow):
                cs = pl.ds(u * L, L)
                for r in range(R):
                    _obuf[r, cs] = _gbuf[r, cs] + pos_v[_srow + r, cs]
            @pl.when(b + 1 < B)
            def _():
                start_gather(b + 1, j)
            pltpu.async_copy(
                obuf, out_hbm.at[b, pl.ds(col0 + srow, R), :], STSEM[j]
            )
        return carry

    lax.fori_loop(0, B, step, 0, unroll=False)

    for j in range(4):
        wait_store(B - 1, j)


@jax.jit
def _emb(x, pix_table, pos_table):
    reps = 8
    pix_rep = jnp.tile(pix_table, (reps, 1))
    off = (jnp.arange(S, dtype=jnp.int32) // SW % reps) * jnp.int32(512)
    x_off = x + off[None, :]
    run = pl.kernel(
        _emb_body,
        out_type=jax.ShapeDtypeStruct((B, S, H), jnp.float32),
        mesh=plsc.VectorSubcoreMesh(core_axis_name="c", subcore_axis_name="s"),
        scratch_types=[
            pltpu.VMEM((B, 128), jnp.int32),
            pltpu.VMEM((SW, H), jnp.float32),
            pltpu.VMEM((R, H), jnp.float32),
            pltpu.VMEM((R, H), jnp.float32),
            pltpu.VMEM((R, H), jnp.float32),
            pltpu.VMEM((R, H), jnp.float32),
            pltpu.VMEM((R, H), jnp.float32),
            pltpu.VMEM((R, H), jnp.float32),
            pltpu.VMEM((R, H), jnp.float32),
            pltpu.VMEM((R, H), jnp.float32),
            pltpu.SemaphoreType.DMA,
            pltpu.SemaphoreType.DMA,
            pltpu.SemaphoreType.DMA,
            pltpu.SemaphoreType.DMA,
            pltpu.SemaphoreType.DMA,
            pltpu.SemaphoreType.DMA,
            pltpu.SemaphoreType.DMA,
            pltpu.SemaphoreType.DMA,
        ],
    )
    return run(x_off, pix_rep, pos_table)


def kernel(x, pix_table, pos_table):
    return _emb(x, pix_table, pos_table)


# speedup vs baseline: 1.0241x; 1.0241x over previous
import jax
import jax.numpy as jnp
from jax import lax
from jax.experimental import pallas as pl
from jax.experimental.pallas import tpu as pltpu
from jax.experimental.pallas import tpu_sc as plsc

NC, NS, L = 2, 16, 16
NW = NC * NS
B, S, H = 32, 1024, 1024
SW = S // NW
R = 8
CPB = SW // R


def _emb_body(x_hbm, pix_hbm, pos_hbm, out_hbm,
              idx_v, pos_v, g0, g1, g2, g3, o0, o1, o2, o3,
              gsem0, gsem1, gsem2, gsem3, stsem0, stsem1, stsem2, stsem3):
    wid = lax.axis_index("s") * NC + lax.axis_index("c")
    col0 = pl.multiple_of(wid * SW, SW)
    xblk = pl.multiple_of((wid // 4) * 128, 128)
    coff = (wid % 4) * SW
    G = (g0, g1, g2, g3)
    O = (o0, o1, o2, o3)
    GSEM = (gsem0, gsem1, gsem2, gsem3)
    STSEM = (stsem0, stsem1, stsem2, stsem3)

    pltpu.sync_copy(x_hbm.at[:, pl.ds(xblk, 128)], idx_v)
    pos_copy = pltpu.async_copy(pos_hbm.at[pl.ds(col0, SW), :], pos_v, stsem0)

    def start_gather(b, j):
        pltpu.async_copy(
            pix_hbm.at[idx_v.at[b, pl.ds(coff + j * R, R)]], G[j], GSEM[j]
        )

    def wait_gather(b, j):
        pltpu.make_async_copy(
            pix_hbm.at[idx_v.at[b, pl.ds(coff + j * R, R)]], G[j], GSEM[j]
        ).wait()

    def wait_store(b, j):
        pltpu.make_async_copy(
            O[j], out_hbm.at[b, pl.ds(col0 + j * R, R), :], STSEM[j]
        ).wait()

    for j in range(CPB):
        start_gather(0, j)
    pos_copy.wait()

    def step(b, carry):
        for j in range(CPB):
            srow = j * R
            gbuf, obuf = G[j], O[j]
            wait_gather(b, j)
            @pl.when(b > 0)
            def _():
                wait_store(b, j)
            @plsc.parallel_loop(0, H // L, step=1, unroll=2)
            def _(u, _obuf=obuf, _gbuf=gbuf, _srow=srow):
                cs = pl.ds(u * L, L)
                for r in range(R):
                    _obuf[r, cs] = _gbuf[r, cs] + pos_v[_srow + r, cs]
            @pl.when(b + 1 < B)
            def _():
                start_gather(b + 1, j)
            pltpu.async_copy(
                obuf, out_hbm.at[b, pl.ds(col0 + srow, R), :], STSEM[j]
            )
        return carry

    lax.fori_loop(0, B, step, 0, unroll=False)

    for j in range(4):
        wait_store(B - 1, j)


@jax.jit
def _emb(x, pix_table, pos_table):
    reps = 4
    pix_rep = jnp.tile(pix_table, (reps, 1))
    off = ((jnp.arange(S, dtype=jnp.int32)[None, :] // SW
            + jnp.arange(B, dtype=jnp.int32)[:, None]) % reps) * jnp.int32(512)
    x_off = x + off
    run = pl.kernel(
        _emb_body,
        out_type=jax.ShapeDtypeStruct((B, S, H), jnp.float32),
        mesh=plsc.VectorSubcoreMesh(core_axis_name="c", subcore_axis_name="s"),
        scratch_types=[
            pltpu.VMEM((B, 128), jnp.int32),
            pltpu.VMEM((SW, H), jnp.float32),
            pltpu.VMEM((R, H), jnp.float32),
            pltpu.VMEM((R, H), jnp.float32),
            pltpu.VMEM((R, H), jnp.float32),
            pltpu.VMEM((R, H), jnp.float32),
            pltpu.VMEM((R, H), jnp.float32),
            pltpu.VMEM((R, H), jnp.float32),
            pltpu.VMEM((R, H), jnp.float32),
            pltpu.VMEM((R, H), jnp.float32),
            pltpu.SemaphoreType.DMA,
            pltpu.SemaphoreType.DMA,
            pltpu.SemaphoreType.DMA,
            pltpu.SemaphoreType.DMA,
            pltpu.SemaphoreType.DMA,
            pltpu.SemaphoreType.DMA,
            pltpu.SemaphoreType.DMA,
            pltpu.SemaphoreType.DMA,
        ],
    )
    return run(x_off, pix_rep, pos_table)


def kernel(x, pix_table, pos_table):
    return _emb(x, pix_table, pos_table)
